# baseline (speedup 1.0000x reference)
import functools

import jax
import jax.numpy as jnp
from jax.experimental import pallas as pl
from jax.experimental.pallas import tpu as pltpu

_BLOCK_ROWS = 4096


def _rowdot_sigmoid_body(x_ref, w_ref, b_ref, o_ref):
    prod = x_ref[...] * w_ref[...]
    s = jnp.sum(prod, axis=1, keepdims=True)
    h = 0.5 * s.T + (0.5 * b_ref[0, 0])
    o_ref[...] = 0.5 * jnp.tanh(h) + 0.5


@functools.partial(jax.jit, static_argnames=("block_rows",))
def _logreg_sigmoid(x, weight, bias, *, block_rows=_BLOCK_ROWS):
    n, f = x.shape
    tm = min(block_rows, n)
    grid = pl.cdiv(n, tm)
    bias2d = bias.reshape(1, 1).astype(jnp.float32)

    x_block_bytes = tm * f * jnp.dtype(x.dtype).itemsize
    vmem_limit = int(min(2 * x_block_bytes + (4 << 20), 60 << 20))

    out = pl.pallas_call(
        _rowdot_sigmoid_body,
        out_shape=jax.ShapeDtypeStruct((1, n), x.dtype),
        grid=(grid,),
        in_specs=[
            pl.BlockSpec((tm, f), lambda i: (i, 0)),
            pl.BlockSpec((1, f), lambda i: (0, 0)),
            pl.BlockSpec((1, 1), lambda i: (0, 0), memory_space=pltpu.SMEM),
        ],
        out_specs=pl.BlockSpec((1, tm), lambda i: (0, i)),
        compiler_params=pltpu.CompilerParams(
            dimension_semantics=("parallel",),
            vmem_limit_bytes=vmem_limit,
        ),
    )(x, weight, bias2d)
    return out.reshape(n, 1)


def kernel(x, weight, bias):
    return _logreg_sigmoid(x, weight, bias)


# speedup vs baseline: 1.0172x; 1.0172x over previous
import functools

import jax
import jax.numpy as jnp
from jax.experimental import pallas as pl
from jax.experimental.pallas import tpu as pltpu

_BLOCK_ROWS = 2048
_HALF = _BLOCK_ROWS // 2


def _rowdot_sigmoid_body(x0_ref, x1_ref, w_ref, b_ref, o_ref):
    w = w_ref[...]
    hb = 0.5 * b_ref[0, 0]
    s0 = jnp.sum(x0_ref[...] * w, axis=1, keepdims=True)
    s1 = jnp.sum(x1_ref[...] * w, axis=1, keepdims=True)
    h0 = 0.5 * s0.T + hb
    h1 = 0.5 * s1.T + hb
    o_ref[:, :_HALF] = 0.5 * jnp.tanh(h0) + 0.5
    o_ref[:, _HALF:] = 0.5 * jnp.tanh(h1) + 0.5


@jax.jit
def _logreg_sigmoid(x, weight, bias):
    n, f = x.shape
    tm = min(_BLOCK_ROWS, n)
    half = tm // 2
    grid = pl.cdiv(n, tm)
    bias2d = bias.reshape(1, 1).astype(jnp.float32)

    x_block_bytes = tm * f * jnp.dtype(x.dtype).itemsize
    vmem_limit = int(min(2 * x_block_bytes + (4 << 20), 60 << 20))

    out = pl.pallas_call(
        _rowdot_sigmoid_body,
        out_shape=jax.ShapeDtypeStruct((1, n), x.dtype),
        grid=(grid,),
        in_specs=[
            pl.BlockSpec((half, f), lambda i: (2 * i, 0)),
            pl.BlockSpec((half, f), lambda i: (2 * i + 1, 0)),
            pl.BlockSpec((1, f), lambda i: (0, 0)),
            pl.BlockSpec((1, 1), lambda i: (0, 0), memory_space=pltpu.SMEM),
        ],
        out_specs=pl.BlockSpec((1, tm), lambda i: (0, i)),
        compiler_params=pltpu.CompilerParams(
            dimension_semantics=("parallel",),
            vmem_limit_bytes=vmem_limit,
        ),
    )(x, x, weight, bias2d)
    return out.reshape(n, 1)


def kernel(x, weight, bias):
    return _logreg_sigmoid(x, weight, bias)
